# baseline (speedup 1.0000x reference)
import jax
import jax.numpy as jnp
from jax import lax
from jax.experimental import pallas as pl
from jax.experimental.pallas import tpu as pltpu

B, M, D, E = 2, 2048, 768, 8
T = B * M
TB = 1024
NB = T // TB
BPB = NB // B
HIGH = lax.Precision.HIGHEST


def _cv2(v):
    mean = jnp.mean(v)
    var1 = jnp.sum((v - mean) ** 2) / (E - 1)
    return var1 / (mean * mean + 1e-10)


def _moe_body(xbf_ref, wg_ref, w1_ref, b1_ref, w2k_ref, b2_ref, yt_ref,
              hw_ref, hb_ref, lng_ref, lnb_ref,
              scores_ref, aux_ref, pred_ref,
              acc_ref, imp_scr, load_scr):
    i = pl.program_id(0)
    b = i // BPB
    xb = xbf_ref[...]
    logits = jnp.dot(xb, wg_ref[...], preferred_element_type=jnp.float32)
    eidx = lax.broadcasted_iota(jnp.int32, (TB, E), 1)
    m1 = jnp.max(logits, axis=1, keepdims=True)
    a1 = jnp.min(jnp.where(logits == m1, eidx, E), axis=1, keepdims=True)
    masked = jnp.where(eidx == a1, -jnp.inf, logits)
    m2 = jnp.max(masked, axis=1, keepdims=True)
    a2 = jnp.min(jnp.where(masked == m2, eidx, E), axis=1, keepdims=True)
    ed = jnp.exp(m2 - m1)
    g1 = 1.0 / (1.0 + ed)
    g2 = ed / (1.0 + ed)
    gates = (jnp.where(eidx == a1, g1, 0.0)
             + jnp.where(eidx == a2, g2, 0.0))
    imp_scr[pl.ds(i, 1), :] = jnp.sum(gates, axis=0)[None, :]
    load_scr[pl.ds(i, 1), :] = jnp.sum((gates > 0.0).astype(jnp.float32),
                                       axis=0)[None, :]

    @pl.when(i == 0)
    def _():
        acc_ref[...] = jnp.zeros_like(acc_ref)

    for e in range(E):
        h = jnp.dot(xb, w1_ref[e], preferred_element_type=jnp.float32)
        h = jnp.maximum(h + b1_ref[e], 0.0)
        vec = jnp.sum(h * gates[:, e:e + 1], axis=0)
        acc_ref[pl.ds(b, 1), e * D:(e + 1) * D] += vec[None, :]

    @pl.when(i == NB - 1)
    def _():
        imp_blk = imp_scr[...]
        load_blk = load_scr[...]
        imp_b = jnp.reshape(imp_blk, (B, BPB, E)).sum(axis=1)
        importance = jnp.sum(imp_blk, axis=0)
        load = jnp.sum(load_blk, axis=0)
        aux = (_cv2(importance) + _cv2(load)) * 0.01
        aux_ref[...] = jnp.reshape(aux, (1, 1))

        mm = jnp.dot(acc_ref[...], w2k_ref[...],
                     preferred_element_type=jnp.float32, precision=HIGH)
        mm = mm + jnp.dot(imp_b, b2_ref[...],
                          preferred_element_type=jnp.float32, precision=HIGH)
        mu = jnp.mean(mm, axis=1, keepdims=True)
        var = jnp.mean((mm - mu) ** 2, axis=1, keepdims=True)
        fin = (mm - mu) * lax.rsqrt(var + 1e-5) * lng_ref[...] + lnb_ref[...]
        out = jnp.dot(fin, hw_ref[...], preferred_element_type=jnp.float32,
                      precision=HIGH) + hb_ref[...]
        scores = jax.nn.sigmoid(out)
        scores_ref[...] = scores
        pred_ref[...] = jnp.reshape(
            jnp.mean((scores - yt_ref[...]) ** 2), (1, 1))


def kernel(mm_embed, task_index, true_y, w_gate, fc1_w, fc1_b, fc2_w, fc2_b,
           head_w, head_b, ln_g, ln_b):
    xbf = mm_embed.reshape(T, D).astype(jnp.bfloat16)
    w1bf = fc1_w.astype(jnp.bfloat16)

    scores, aux, pred = pl.pallas_call(
        _moe_body,
        grid=(NB,),
        in_specs=[
            pl.BlockSpec((TB, D), lambda i: (i, 0)),
            pl.BlockSpec((D, E), lambda i: (0, 0)),
            pl.BlockSpec((E, D, D), lambda i: (0, 0, 0)),
            pl.BlockSpec((E, 1, D), lambda i: (0, 0, 0)),
            pl.BlockSpec((E * D, D), lambda i: (0, 0)),
            pl.BlockSpec((E, D), lambda i: (0, 0)),
            pl.BlockSpec((B, 1), lambda i: (0, 0)),
            pl.BlockSpec((D, 1), lambda i: (0, 0)),
            pl.BlockSpec((1, 1), lambda i: (0, 0)),
            pl.BlockSpec((1, D), lambda i: (0, 0)),
            pl.BlockSpec((1, D), lambda i: (0, 0)),
        ],
        out_specs=[
            pl.BlockSpec((B, 1), lambda i: (0, 0)),
            pl.BlockSpec((1, 1), lambda i: (0, 0)),
            pl.BlockSpec((1, 1), lambda i: (0, 0)),
        ],
        out_shape=[
            jax.ShapeDtypeStruct((B, 1), jnp.float32),
            jax.ShapeDtypeStruct((1, 1), jnp.float32),
            jax.ShapeDtypeStruct((1, 1), jnp.float32),
        ],
        scratch_shapes=[
            pltpu.VMEM((B, E * D), jnp.float32),
            pltpu.VMEM((NB, E), jnp.float32),
            pltpu.VMEM((NB, E), jnp.float32),
        ],
    )(xbf, w_gate.astype(jnp.bfloat16), w1bf, fc1_b.reshape(E, 1, D),
      fc2_w.reshape(E * D, D), fc2_b, true_y,
      head_w, head_b.reshape(1, 1), ln_g.reshape(1, D), ln_b.reshape(1, D))

    return (scores, aux.reshape(()), pred.reshape(()))


# speedup vs baseline: 1.1854x; 1.1854x over previous
import jax
import jax.numpy as jnp
from jax import lax
from jax.experimental import pallas as pl
from jax.experimental.pallas import tpu as pltpu

B, M, D, E = 2, 2048, 768, 8
T = B * M
TB = 1024
NB = T // TB
BPB = NB // B
HIGH = lax.Precision.HIGHEST


def _cv2(v):
    mean = jnp.mean(v)
    var1 = jnp.sum((v - mean) ** 2) / (E - 1)
    return var1 / (mean * mean + 1e-10)


def _moe_body(xbf_ref, wg_ref, w1_ref, b1_ref, w2k_ref, b2_ref, yt_ref,
              hw_ref, hb_ref, lng_ref, lnb_ref,
              scores_ref, aux_ref, pred_ref,
              acc_ref, imp_scr, load_scr):
    i = pl.program_id(0)
    b = i // BPB
    xf = xbf_ref[...]
    xb = xf.astype(jnp.bfloat16)
    logits = jnp.dot(xb, wg_ref[...], preferred_element_type=jnp.float32)
    eidx = lax.broadcasted_iota(jnp.int32, (TB, E), 1)
    m1 = jnp.max(logits, axis=1, keepdims=True)
    a1 = jnp.min(jnp.where(logits == m1, eidx, E), axis=1, keepdims=True)
    masked = jnp.where(eidx == a1, -jnp.inf, logits)
    m2 = jnp.max(masked, axis=1, keepdims=True)
    a2 = jnp.min(jnp.where(masked == m2, eidx, E), axis=1, keepdims=True)
    ed = jnp.exp(m2 - m1)
    g1 = 1.0 / (1.0 + ed)
    g2 = ed / (1.0 + ed)
    gates = (jnp.where(eidx == a1, g1, 0.0)
             + jnp.where(eidx == a2, g2, 0.0))
    imp_scr[pl.ds(i, 1), :] = jnp.sum(gates, axis=0)[None, :]
    load_scr[pl.ds(i, 1), :] = jnp.sum((gates > 0.0).astype(jnp.float32),
                                       axis=0)[None, :]

    @pl.when(i == 0)
    def _():
        acc_ref[...] = jnp.zeros_like(acc_ref)

    for e in range(E):
        h = jnp.dot(xf, w1_ref[e], preferred_element_type=jnp.float32,
                    precision=lax.Precision.DEFAULT)
        h = jnp.maximum(h + b1_ref[e], 0.0)
        vec = jnp.sum(h * gates[:, e:e + 1], axis=0)
        acc_ref[pl.ds(b, 1), e * D:(e + 1) * D] += vec[None, :]

    @pl.when(i == NB - 1)
    def _():
        imp_blk = imp_scr[...]
        load_blk = load_scr[...]
        imp_b = jnp.reshape(imp_blk, (B, BPB, E)).sum(axis=1)
        importance = jnp.sum(imp_blk, axis=0)
        load = jnp.sum(load_blk, axis=0)
        aux = (_cv2(importance) + _cv2(load)) * 0.01
        aux_ref[...] = jnp.reshape(aux, (1, 1))

        mm = jnp.dot(acc_ref[...], w2k_ref[...],
                     preferred_element_type=jnp.float32, precision=HIGH)
        mm = mm + jnp.dot(imp_b, b2_ref[...],
                          preferred_element_type=jnp.float32, precision=HIGH)
        mu = jnp.mean(mm, axis=1, keepdims=True)
        var = jnp.mean((mm - mu) ** 2, axis=1, keepdims=True)
        fin = (mm - mu) * lax.rsqrt(var + 1e-5) * lng_ref[...] + lnb_ref[...]
        out = jnp.dot(fin, hw_ref[...], preferred_element_type=jnp.float32,
                      precision=HIGH) + hb_ref[...]
        scores = jax.nn.sigmoid(out)
        scores_ref[...] = scores
        pred_ref[...] = jnp.reshape(
            jnp.mean((scores - yt_ref[...]) ** 2), (1, 1))


def kernel(mm_embed, task_index, true_y, w_gate, fc1_w, fc1_b, fc2_w, fc2_b,
           head_w, head_b, ln_g, ln_b):
    xbf = mm_embed.reshape(T, D)
    w1bf = fc1_w

    scores, aux, pred = pl.pallas_call(
        _moe_body,
        grid=(NB,),
        in_specs=[
            pl.BlockSpec((TB, D), lambda i: (i, 0)),
            pl.BlockSpec((D, E), lambda i: (0, 0)),
            pl.BlockSpec((E, D, D), lambda i: (0, 0, 0)),
            pl.BlockSpec((E, 1, D), lambda i: (0, 0, 0)),
            pl.BlockSpec((E * D, D), lambda i: (0, 0)),
            pl.BlockSpec((E, D), lambda i: (0, 0)),
            pl.BlockSpec((B, 1), lambda i: (0, 0)),
            pl.BlockSpec((D, 1), lambda i: (0, 0)),
            pl.BlockSpec((1, 1), lambda i: (0, 0)),
            pl.BlockSpec((1, D), lambda i: (0, 0)),
            pl.BlockSpec((1, D), lambda i: (0, 0)),
        ],
        out_specs=[
            pl.BlockSpec((B, 1), lambda i: (0, 0)),
            pl.BlockSpec((1, 1), lambda i: (0, 0)),
            pl.BlockSpec((1, 1), lambda i: (0, 0)),
        ],
        out_shape=[
            jax.ShapeDtypeStruct((B, 1), jnp.float32),
            jax.ShapeDtypeStruct((1, 1), jnp.float32),
            jax.ShapeDtypeStruct((1, 1), jnp.float32),
        ],
        scratch_shapes=[
            pltpu.VMEM((B, E * D), jnp.float32),
            pltpu.VMEM((NB, E), jnp.float32),
            pltpu.VMEM((NB, E), jnp.float32),
        ],
    )(xbf, w_gate.astype(jnp.bfloat16), w1bf, fc1_b.reshape(E, 1, D),
      fc2_w.reshape(E * D, D), fc2_b, true_y,
      head_w, head_b.reshape(1, 1), ln_g.reshape(1, D), ln_b.reshape(1, D))

    return (scores, aux.reshape(()), pred.reshape(()))


# speedup vs baseline: 1.2117x; 1.0222x over previous
import jax
import jax.numpy as jnp
from jax import lax
from jax.experimental import pallas as pl
from jax.experimental.pallas import tpu as pltpu

B, M, D, E = 2, 2048, 768, 8
T = B * M
TB = 2048
NB = T // TB
BPB = NB // B
HIGH = lax.Precision.HIGHEST


def _cv2(v):
    mean = jnp.mean(v)
    var1 = jnp.sum((v - mean) ** 2) / (E - 1)
    return var1 / (mean * mean + 1e-10)


def _moe_body(x_ref, wg_ref, w1_ref, b1_ref, w2_ref, b2_ref, yt_ref,
              hw_ref, hb_ref, lng_ref, lnb_ref,
              scores_ref, aux_ref, pred_ref,
              gates_scr, imp_scr, load_scr, fold_scr, mm_scr):
    e = pl.program_id(0)
    i = pl.program_id(1)
    b = i // BPB
    xf = x_ref[pl.ds(i * TB, TB), :]

    @pl.when(jnp.logical_and(e == 0, i == 0))
    def _():
        mm_scr[...] = jnp.zeros_like(mm_scr)

    @pl.when(e == 0)
    def _():
        xb = xf.astype(jnp.bfloat16)
        logits = jnp.dot(xb, wg_ref[...].astype(jnp.bfloat16),
                         preferred_element_type=jnp.float32)
        eidx = lax.broadcasted_iota(jnp.int32, (TB, E), 1)
        m1 = jnp.max(logits, axis=1, keepdims=True)
        a1 = jnp.min(jnp.where(logits == m1, eidx, E), axis=1, keepdims=True)
        masked = jnp.where(eidx == a1, -jnp.inf, logits)
        m2 = jnp.max(masked, axis=1, keepdims=True)
        a2 = jnp.min(jnp.where(masked == m2, eidx, E), axis=1, keepdims=True)
        ed = jnp.exp(m2 - m1)
        g1 = 1.0 / (1.0 + ed)
        g2 = ed / (1.0 + ed)
        gates = (jnp.where(eidx == a1, g1, 0.0)
                 + jnp.where(eidx == a2, g2, 0.0))
        gates_scr[pl.ds(i * TB, TB), :] = gates
        imp_scr[pl.ds(i, 1), :] = jnp.sum(gates, axis=0)[None, :]
        load_scr[pl.ds(i, 1), :] = jnp.sum((gates > 0.0).astype(jnp.float32),
                                           axis=0)[None, :]

    @pl.when(i == 0)
    def _():
        fold_scr[...] = jnp.zeros_like(fold_scr)

    h = jnp.dot(xf, w1_ref[0], preferred_element_type=jnp.float32,
                precision=lax.Precision.DEFAULT)
    h = jnp.maximum(h + b1_ref[0], 0.0)
    gall = gates_scr[pl.ds(i * TB, TB), :]
    sel = lax.broadcasted_iota(jnp.int32, (TB, E), 1) == e
    gcol = jnp.sum(jnp.where(sel, gall, 0.0), axis=1, keepdims=True)
    vec = jnp.sum(h * gcol, axis=0)
    fold_scr[pl.ds(b, 1), :] += vec[None, :]

    @pl.when(i == NB - 1)
    def _():
        mm_scr[0:B, :] += jnp.dot(fold_scr[0:B, :], w2_ref[0],
                                  preferred_element_type=jnp.float32,
                                  precision=lax.Precision.DEFAULT)

    @pl.when(jnp.logical_and(e == E - 1, i == NB - 1))
    def _():
        imp_blk = imp_scr[...]
        load_blk = load_scr[...]
        imp_b = jnp.reshape(imp_blk, (B, BPB, E)).sum(axis=1)
        importance = jnp.sum(imp_blk, axis=0)
        load = jnp.sum(load_blk, axis=0)
        aux = (_cv2(importance) + _cv2(load)) * 0.01
        aux_ref[...] = jnp.reshape(aux, (1, 1))

        mm = mm_scr[0:B, :] + jnp.dot(imp_b, b2_ref[...],
                                      preferred_element_type=jnp.float32,
                                      precision=HIGH)
        mu = jnp.mean(mm, axis=1, keepdims=True)
        var = jnp.mean((mm - mu) ** 2, axis=1, keepdims=True)
        fin = (mm - mu) * lax.rsqrt(var + 1e-5) * lng_ref[...] + lnb_ref[...]
        out = jnp.dot(fin, hw_ref[...], preferred_element_type=jnp.float32,
                      precision=HIGH) + hb_ref[...]
        scores = jax.nn.sigmoid(out)
        scores_ref[...] = scores
        pred_ref[...] = jnp.reshape(
            jnp.mean((scores - yt_ref[...]) ** 2), (1, 1))


def kernel(mm_embed, task_index, true_y, w_gate, fc1_w, fc1_b, fc2_w, fc2_b,
           head_w, head_b, ln_g, ln_b):
    x = mm_embed.reshape(T, D)

    scores, aux, pred = pl.pallas_call(
        _moe_body,
        grid=(E, NB),
        in_specs=[
            pl.BlockSpec((T, D), lambda e, i: (0, 0)),
            pl.BlockSpec((D, E), lambda e, i: (0, 0)),
            pl.BlockSpec((1, D, D), lambda e, i: (e, 0, 0)),
            pl.BlockSpec((1, 1, D), lambda e, i: (e, 0, 0)),
            pl.BlockSpec((1, D, D), lambda e, i: (e, 0, 0)),
            pl.BlockSpec((E, D), lambda e, i: (0, 0)),
            pl.BlockSpec((B, 1), lambda e, i: (0, 0)),
            pl.BlockSpec((D, 1), lambda e, i: (0, 0)),
            pl.BlockSpec((1, 1), lambda e, i: (0, 0)),
            pl.BlockSpec((1, D), lambda e, i: (0, 0)),
            pl.BlockSpec((1, D), lambda e, i: (0, 0)),
        ],
        out_specs=[
            pl.BlockSpec((B, 1), lambda e, i: (0, 0)),
            pl.BlockSpec((1, 1), lambda e, i: (0, 0)),
            pl.BlockSpec((1, 1), lambda e, i: (0, 0)),
        ],
        out_shape=[
            jax.ShapeDtypeStruct((B, 1), jnp.float32),
            jax.ShapeDtypeStruct((1, 1), jnp.float32),
            jax.ShapeDtypeStruct((1, 1), jnp.float32),
        ],
        scratch_shapes=[
            pltpu.VMEM((T, E), jnp.float32),
            pltpu.VMEM((NB, E), jnp.float32),
            pltpu.VMEM((NB, E), jnp.float32),
            pltpu.VMEM((8, D), jnp.float32),
            pltpu.VMEM((8, D), jnp.float32),
        ],
    )(x, w_gate, fc1_w, fc1_b.reshape(E, 1, D), fc2_w, fc2_b, true_y,
      head_w, head_b.reshape(1, 1), ln_g.reshape(1, D), ln_b.reshape(1, D))

    return (scores, aux.reshape(()), pred.reshape(()))
